# Initial kernel scaffold; baseline (speedup 1.0000x reference)
import jax
import jax.numpy as jnp
from jax.experimental import pallas as pl


def kernel(x, W):
    raise NotImplementedError("write your pallas kernel here")


# speedup vs baseline: 4.3305x; 4.3305x over previous
import functools

import jax
import jax.numpy as jnp
from jax import lax
from jax.experimental import pallas as pl
from jax.experimental.pallas import tpu as pltpu
from jax.experimental.pallas import tpu_sc as plsc

_VOCAB = 1000
_D = 10
_ROWS, _COLS = 16384, 200
_N = _ROWS * _COLS
_NC, _NS = 2, 16
_NW = _NC * _NS
_PER_W = _N // _NW
_CHUNK = 2048
_N_CHUNKS = _PER_W // _CHUNK
_GROUPS = _CHUNK // 16


def _sc_body(x_hbm, w_hbm, out_hbm, table_v, idx_v, out_v):
    wid = lax.axis_index("s") * _NC + lax.axis_index("c")
    pltpu.sync_copy(w_hbm, table_v)
    lane10 = lax.iota(jnp.int32, 16) * _D

    def chunk_body(k, carry):
        base = wid * _PER_W + k * _CHUNK
        pltpu.sync_copy(x_hbm.at[pl.ds(base, _CHUNK)], idx_v)

        def group(g, carry):
            iv = idx_v[pl.ds(g * 16, 16)]
            gbase = iv * _D
            obase = g * (16 * _D) + lane10
            for d in range(_D):
                v = plsc.load_gather(table_v, [gbase + d])
                plsc.store_scatter(out_v, [obase + d], v)
            return carry

        lax.fori_loop(0, _GROUPS, group, 0)
        pltpu.sync_copy(out_v, out_hbm.at[pl.ds(base * _D, _CHUNK * _D)])
        return carry

    lax.fori_loop(0, _N_CHUNKS, chunk_body, 0)


@jax.jit
def kernel(x, W):
    x_flat = x.reshape(-1)
    w_flat = W.reshape(-1)
    mesh = plsc.VectorSubcoreMesh(
        core_axis_name="c", subcore_axis_name="s",
        num_cores=_NC, num_subcores=_NS,
    )
    out = pl.kernel(
        _sc_body,
        out_type=jax.ShapeDtypeStruct((_N * _D,), jnp.float32),
        mesh=mesh,
        compiler_params=pltpu.CompilerParams(needs_layout_passes=False),
        scratch_types=[
            pltpu.VMEM((_VOCAB * _D,), jnp.float32),
            pltpu.VMEM((_CHUNK,), jnp.int32),
            pltpu.VMEM((_CHUNK * _D,), jnp.float32),
        ],
    )(x_flat, w_flat)
    return out.reshape(_ROWS, _COLS, _D)


# speedup vs baseline: 4.6795x; 1.0806x over previous
import jax
import jax.numpy as jnp
from jax import lax
from jax.experimental import pallas as pl
from jax.experimental.pallas import tpu as pltpu
from jax.experimental.pallas import tpu_sc as plsc

_VOCAB = 1000
_D = 10
_ROWS, _COLS = 16384, 200
_N = _ROWS * _COLS
_NC, _NS = 2, 16
_NW = _NC * _NS
_PER_W = _N // _NW
_CHUNK = 4096
_N_CHUNKS = _PER_W // _CHUNK
_GROUPS = _CHUNK // 16
_UNROLL = 4


def _sc_body(x_hbm, w_hbm, out_hbm, table_v, idx_v, out_v):
    wid = lax.axis_index("s") * _NC + lax.axis_index("c")
    wbase = wid * _PER_W
    pltpu.sync_copy(w_hbm, table_v)
    lane10 = lax.iota(jnp.int32, 16) * _D

    @pl.loop(0, _N_CHUNKS)
    def _chunks(k):
        base = wbase + k * _CHUNK
        pltpu.sync_copy(x_hbm.at[pl.ds(base, _CHUNK)], idx_v)

        @plsc.parallel_loop(0, _GROUPS, unroll=_UNROLL)
        def _group(g):
            iv = idx_v[pl.ds(g * 16, 16)]
            gbase = iv * _D
            obase = g * (16 * _D) + lane10
            for d in range(_D):
                v = plsc.load_gather(table_v, [gbase + d])
                plsc.store_scatter(out_v, [obase + d], v)

        pltpu.sync_copy(out_v, out_hbm.at[pl.ds(base * _D, _CHUNK * _D)])


@jax.jit
def kernel(x, W):
    x_flat = x.reshape(-1)
    w_flat = W.reshape(-1)
    mesh = plsc.VectorSubcoreMesh(
        core_axis_name="c", subcore_axis_name="s",
        num_cores=_NC, num_subcores=_NS,
    )
    out = pl.kernel(
        _sc_body,
        out_type=jax.ShapeDtypeStruct((_N * _D,), jnp.float32),
        mesh=mesh,
        compiler_params=pltpu.CompilerParams(needs_layout_passes=False),
        scratch_types=[
            pltpu.VMEM((_VOCAB * _D,), jnp.float32),
            pltpu.VMEM((_CHUNK,), jnp.int32),
            pltpu.VMEM((_CHUNK * _D,), jnp.float32),
        ],
    )(x_flat, w_flat)
    return out.reshape(_ROWS, _COLS, _D)


# speedup vs baseline: 93.2924x; 19.9366x over previous
import jax
import jax.numpy as jnp
from jax import lax
from jax.experimental import pallas as pl
from jax.experimental.pallas import tpu as pltpu
from jax.experimental.pallas import tpu_sc as plsc

_VOCAB = 1000
_D = 10
_ROWS, _COLS = 16384, 200
_N = _ROWS * _COLS
_NC, _NS = 2, 16
_NW = _NC * _NS
_PER_W = _N // _NW
_CHUNK = 4096
_N_CHUNKS = _PER_W // _CHUNK
_GROUPS = _CHUNK // 16
_UNROLL = 4


def _sc_body(x_hbm, wt_hbm, out_hbm, table_v, idx_v, out_v, sem_out):
    wid = lax.axis_index("s") * _NC + lax.axis_index("c")
    wbase = wid * _PER_W
    pltpu.sync_copy(wt_hbm, table_v)

    @pl.loop(0, _N_CHUNKS)
    def _chunks(k):
        base = wbase + k * _CHUNK
        pltpu.sync_copy(x_hbm.at[pl.ds(base, _CHUNK)], idx_v)

        @plsc.parallel_loop(0, _GROUPS, unroll=_UNROLL)
        def _group(g):
            iv = idx_v[pl.ds(g * 16, 16)]
            for d in range(_D):
                v = plsc.load_gather(table_v, [iv + (d * _VOCAB)])
                out_v[pl.ds(d * _CHUNK + g * 16, 16)] = v

        copies = [
            pltpu.make_async_copy(
                out_v.at[pl.ds(d * _CHUNK, _CHUNK)],
                out_hbm.at[pl.ds(d * _N + base, _CHUNK)],
                sem_out)
            for d in range(_D)
        ]
        for c in copies:
            c.start()
        for c in copies:
            c.wait()


@jax.jit
def kernel(x, W):
    xp = x.T.reshape(_COLS // 8, 8, _ROWS // 128, 128)
    x_flat = xp.transpose(0, 2, 1, 3).reshape(-1)
    wt_flat = W.T.reshape(-1)
    mesh = plsc.VectorSubcoreMesh(
        core_axis_name="c", subcore_axis_name="s",
        num_cores=_NC, num_subcores=_NS,
    )
    out = pl.kernel(
        _sc_body,
        out_type=jax.ShapeDtypeStruct((_D * _N,), jnp.float32),
        mesh=mesh,
        compiler_params=pltpu.CompilerParams(needs_layout_passes=False),
        scratch_types=[
            pltpu.VMEM((_VOCAB * _D,), jnp.float32),
            pltpu.VMEM((_CHUNK,), jnp.int32),
            pltpu.VMEM((_CHUNK * _D,), jnp.float32),
            pltpu.SemaphoreType.DMA,
        ],
    )(x_flat, wt_flat)
    t = out.reshape(_D, _COLS // 8, _ROWS // 128, 8, 128)
    return t.transpose(2, 4, 1, 3, 0).reshape(_ROWS, _COLS, _D)


# speedup vs baseline: 153.7471x; 1.6480x over previous
import jax
import jax.numpy as jnp
from jax import lax
from jax.experimental import pallas as pl
from jax.experimental.pallas import tpu as pltpu
from jax.experimental.pallas import tpu_sc as plsc

_VOCAB = 1000
_D = 10
_ROWS, _COLS = 16384, 200
_N = _ROWS * _COLS
_NC, _NS = 2, 16
_NW = _NC * _NS
_PER_W = _N // _NW
_CHUNK = 2048
_N_CHUNKS = _PER_W // _CHUNK
_GROUPS = _CHUNK // 16
_UNROLL = 4


def _sc_body(x_hbm, wt_hbm, out_hbm, table_v,
             idx0, idx1, out0, out1, sin0, sin1, sout0, sout1):
    wid = lax.axis_index("s") * _NC + lax.axis_index("c")
    wbase = wid * _PER_W
    pltpu.sync_copy(wt_hbm, table_v)

    idx_bufs = (idx0, idx1)
    out_bufs = (out0, out1)
    sins = (sin0, sin1)
    souts = (sout0, sout1)

    def idx_copy(k, b):
        return pltpu.make_async_copy(
            x_hbm.at[pl.ds(wbase + k * _CHUNK, _CHUNK)], idx_bufs[b], sins[b])

    def out_copies(k, b):
        base = wbase + k * _CHUNK
        return [
            pltpu.make_async_copy(
                out_bufs[b].at[pl.ds(d * _CHUNK, _CHUNK)],
                out_hbm.at[pl.ds(d * _N + base, _CHUNK)],
                souts[b])
            for d in range(_D)
        ]

    def compute(b):
        idx_v = idx_bufs[b]
        out_v = out_bufs[b]

        @plsc.parallel_loop(0, _GROUPS, unroll=_UNROLL)
        def _group(g):
            iv = idx_v[pl.ds(g * 16, 16)]
            for d in range(_D):
                v = plsc.load_gather(table_v, [iv + (d * _VOCAB)])
                out_v[pl.ds(d * _CHUNK + g * 16, 16)] = v

    idx_copy(0, 0).start()
    idx_copy(1, 1).start()
    for b in range(2):
        idx_copy(b, b).wait()
        compute(b)
        for c in out_copies(b, b):
            c.start()
        idx_copy(b + 2, b).start()

    @pl.loop(2, _N_CHUNKS, step=2)
    def _chunks(k0):
        for b in range(2):
            k = k0 + b
            idx_copy(k, b).wait()
            for c in out_copies(k, b):
                c.wait()
            compute(b)
            for c in out_copies(k, b):
                c.start()
            k_next = jnp.minimum(k + 2, _N_CHUNKS - 1)
            idx_copy(k_next, b).start()

    for b in range(2):
        idx_copy(0, b).wait()
        for c in out_copies(0, b):
            c.wait()


@jax.jit
def kernel(x, W):
    xp = x.T.reshape(_COLS // 8, 8, _ROWS // 128, 128)
    x_flat = xp.transpose(0, 2, 1, 3).reshape(-1)
    wt_flat = W.T.reshape(-1)
    mesh = plsc.VectorSubcoreMesh(
        core_axis_name="c", subcore_axis_name="s",
        num_cores=_NC, num_subcores=_NS,
    )
    out = pl.kernel(
        _sc_body,
        out_type=jax.ShapeDtypeStruct((_D * _N,), jnp.float32),
        mesh=mesh,
        compiler_params=pltpu.CompilerParams(needs_layout_passes=False),
        scratch_types=[
            pltpu.VMEM((_VOCAB * _D,), jnp.float32),
            pltpu.VMEM((_CHUNK,), jnp.int32),
            pltpu.VMEM((_CHUNK,), jnp.int32),
            pltpu.VMEM((_CHUNK * _D,), jnp.float32),
            pltpu.VMEM((_CHUNK * _D,), jnp.float32),
            pltpu.SemaphoreType.DMA,
            pltpu.SemaphoreType.DMA,
            pltpu.SemaphoreType.DMA,
            pltpu.SemaphoreType.DMA,
        ],
    )(x_flat, wt_flat)
    t = out.reshape(_D, _COLS // 8, _ROWS // 128, 8, 128)
    return t.transpose(2, 4, 1, 3, 0).reshape(_ROWS, _COLS, _D)


# speedup vs baseline: 161.4222x; 1.0499x over previous
import jax
import jax.numpy as jnp
from jax import lax
from jax.experimental import pallas as pl
from jax.experimental.pallas import tpu as pltpu
from jax.experimental.pallas import tpu_sc as plsc

_VOCAB = 1000
_D = 10
_ROWS, _COLS = 16384, 200
_N = _ROWS * _COLS
_NC, _NS = 2, 16
_NW = _NC * _NS
_PER_W = _N // _NW
_CHUNK = 5120
_N_CHUNKS = _PER_W // _CHUNK
_GROUPS = _CHUNK // 16
_UNROLL = 4


def _sc_body(x_hbm, wt_hbm, out_hbm, table_v,
             idx0, idx1, out0, out1, sin0, sin1, sout0, sout1):
    wid = lax.axis_index("s") * _NC + lax.axis_index("c")
    wbase = wid * _PER_W
    pltpu.sync_copy(wt_hbm, table_v)

    idx_bufs = (idx0, idx1)
    out_bufs = (out0, out1)
    sins = (sin0, sin1)
    souts = (sout0, sout1)

    def idx_copy(k, b):
        return pltpu.make_async_copy(
            x_hbm.at[pl.ds(wbase + k * _CHUNK, _CHUNK)], idx_bufs[b], sins[b])

    def out_copies(k, b):
        base = wbase + k * _CHUNK
        return [
            pltpu.make_async_copy(
                out_bufs[b].at[pl.ds(d * _CHUNK, _CHUNK)],
                out_hbm.at[pl.ds(d * _N + base, _CHUNK)],
                souts[b])
            for d in range(_D)
        ]

    def compute(b):
        idx_v = idx_bufs[b]
        out_v = out_bufs[b]

        @plsc.parallel_loop(0, _GROUPS, unroll=_UNROLL)
        def _group(g):
            iv = idx_v[pl.ds(g * 16, 16)]
            for d in range(_D):
                v = plsc.load_gather(table_v, [iv + (d * _VOCAB)])
                out_v[pl.ds(d * _CHUNK + g * 16, 16)] = v

    idx_copy(0, 0).start()
    idx_copy(1, 1).start()
    for b in range(2):
        idx_copy(b, b).wait()
        compute(b)
        for c in out_copies(b, b):
            c.start()
        idx_copy(b + 2, b).start()

    @pl.loop(2, _N_CHUNKS, step=2)
    def _chunks(k0):
        for b in range(2):
            k = k0 + b
            idx_copy(k, b).wait()
            for c in out_copies(k, b):
                c.wait()
            compute(b)
            for c in out_copies(k, b):
                c.start()
            k_next = jnp.minimum(k + 2, _N_CHUNKS - 1)
            idx_copy(k_next, b).start()

    for b in range(2):
        idx_copy(0, b).wait()
        for c in out_copies(0, b):
            c.wait()


@jax.jit
def kernel(x, W):
    xp = x.T.reshape(_COLS // 8, 8, _ROWS // 128, 128)
    x_flat = xp.transpose(0, 2, 1, 3).reshape(-1)
    wt_flat = W.T.reshape(-1)
    mesh = plsc.VectorSubcoreMesh(
        core_axis_name="c", subcore_axis_name="s",
        num_cores=_NC, num_subcores=_NS,
    )
    out = pl.kernel(
        _sc_body,
        out_type=jax.ShapeDtypeStruct((_D * _N,), jnp.float32),
        mesh=mesh,
        compiler_params=pltpu.CompilerParams(needs_layout_passes=False),
        scratch_types=[
            pltpu.VMEM((_VOCAB * _D,), jnp.float32),
            pltpu.VMEM((_CHUNK,), jnp.int32),
            pltpu.VMEM((_CHUNK,), jnp.int32),
            pltpu.VMEM((_CHUNK * _D,), jnp.float32),
            pltpu.VMEM((_CHUNK * _D,), jnp.float32),
            pltpu.SemaphoreType.DMA,
            pltpu.SemaphoreType.DMA,
            pltpu.SemaphoreType.DMA,
            pltpu.SemaphoreType.DMA,
        ],
    )(x_flat, wt_flat)
    t = out.reshape(_D, _COLS // 8, _ROWS // 128, 8, 128)
    return t.transpose(2, 4, 1, 3, 0).reshape(_ROWS, _COLS, _D)
